# baseline (speedup 1.0000x reference)
import functools

import jax
import jax.numpy as jnp
from jax import lax
from jax.experimental import pallas as pl
from jax.experimental.pallas import tpu as pltpu
from jax.experimental.pallas import tpu_sc as plsc

N_TOK = 8192
D = 256
K = 8192
R = 256
GRID = N_TOK // R
INV_TEMP = 100.0

_NC = 2
_NS = 16
_NW = _NC * _NS
_CHUNK = 128
_N_CHUNKS = N_TOK // _CHUNK
_CH_PER_W = _N_CHUNKS // _NW


def _vq_body(z_ref, emb_ref, idx_ref, vq_ref, commit_ref, ent_ref,
             en_s, colsum, vq_acc, ent_acc):
    i = pl.program_id(0)

    @pl.when(i == 0)
    def _init():
        colsum[...] = jnp.zeros_like(colsum)
        vq_acc[...] = jnp.zeros_like(vq_acc)
        ent_acc[...] = jnp.zeros_like(ent_acc)
        eb = emb_ref[...]
        enrm = jnp.sqrt(jnp.sum(eb * eb, axis=1, keepdims=True))
        en_s[...] = eb / (enrm + 1e-12)

    zb = z_ref[...]
    nrm = jnp.sqrt(jnp.sum(zb * zb, axis=1, keepdims=True))
    zn = zb / (nrm + 1e-12)
    zsq = jnp.sum(zn * zn, axis=1, keepdims=True)

    dots = lax.dot_general(zn, en_s[...],
                           (((1,), (1,)), ((), ())),
                           preferred_element_type=jnp.float32)
    c = 2.0 - 2.0 * dots
    mc = jnp.min(c, axis=1, keepdims=True)
    kio = lax.broadcasted_iota(jnp.int32, (R, K), 1)
    idx = jnp.min(jnp.where(c == mc, kio, K), axis=1)
    idx_ref[...] = idx.astype(jnp.int32).reshape(R, 1)

    t = (mc - c) * INV_TEMP
    p = jnp.exp(t)
    ones_k = jnp.ones((K, 1), jnp.float32)
    zsum = lax.dot_general(p, ones_k, (((1,), (0,)), ((), ())),
                           preferred_element_type=jnp.float32)
    invz_t = jnp.transpose(1.0 / zsum)
    colsum[...] += lax.dot_general(invz_t, p, (((1,), (0,)), ((), ())),
                                   preferred_element_type=jnp.float32)
    w = lax.dot_general(p * t, ones_k, (((1,), (0,)), ((), ())),
                        preferred_element_type=jnp.float32)
    ent_acc[...] += jnp.sum(w / zsum - jnp.log(zsum),
                            axis=(0, 1), keepdims=True)
    vq_acc[...] += jnp.sum(zsq - 1.0 + mc, axis=(0, 1), keepdims=True)

    @pl.when(i == GRID - 1)
    def _fin():
        vq = vq_acc[...] * (1.0 / (N_TOK * D))
        vq_ref[...] = vq
        commit_ref[...] = 0.25 * vq
        ap = colsum[...] * (1.0 / N_TOK)
        avg_ent = -jnp.sum(ap * jnp.log(ap + 1e-5),
                           axis=(0, 1), keepdims=True)
        samp_ent = -ent_acc[...] * (1.0 / N_TOK)
        ent_ref[...] = 0.1 * (samp_ent - avg_ent)


def _vq_main(z_flat, emb):
    return pl.pallas_call(
        _vq_body,
        grid=(GRID,),
        in_specs=[
            pl.BlockSpec((R, D), lambda i: (i, 0)),
            pl.BlockSpec((K, D), lambda i: (0, 0)),
        ],
        out_specs=[
            pl.BlockSpec((R, 1), lambda i: (i, 0)),
            pl.BlockSpec((1, 1), lambda i: (0, 0)),
            pl.BlockSpec((1, 1), lambda i: (0, 0)),
            pl.BlockSpec((1, 1), lambda i: (0, 0)),
        ],
        out_shape=[
            jax.ShapeDtypeStruct((N_TOK, 1), jnp.int32),
            jax.ShapeDtypeStruct((1, 1), jnp.float32),
            jax.ShapeDtypeStruct((1, 1), jnp.float32),
            jax.ShapeDtypeStruct((1, 1), jnp.float32),
        ],
        scratch_shapes=[
            pltpu.VMEM((K, D), jnp.float32),
            pltpu.VMEM((1, K), jnp.float32),
            pltpu.VMEM((1, 1), jnp.float32),
            pltpu.VMEM((1, 1), jnp.float32),
        ],
        compiler_params=pltpu.CompilerParams(
            dimension_semantics=("arbitrary",)),
    )(z_flat, emb)


def _gather_rows(table, idx):
    mesh = plsc.VectorSubcoreMesh(core_axis_name="c", subcore_axis_name="s")

    @functools.partial(
        pl.kernel,
        out_type=jax.ShapeDtypeStruct((_N_CHUNKS, _CHUNK, D), jnp.float32),
        mesh=mesh,
        scratch_types=[
            pltpu.VMEM((_CH_PER_W, _CHUNK), jnp.int32),
            pltpu.VMEM((_CH_PER_W, _CHUNK, D), jnp.float32),
            pltpu.SemaphoreType.DMA,
        ],
    )
    def gk(table_hbm, idx_hbm, out_hbm, idx_v, rows_v, sem):
        wid = lax.axis_index("s") * _NC + lax.axis_index("c")
        base = wid * _CH_PER_W
        pltpu.sync_copy(idx_hbm.at[pl.ds(base, _CH_PER_W)], idx_v)
        for j in range(_CH_PER_W):
            pltpu.async_copy(table_hbm.at[idx_v.at[j]], rows_v.at[j],
                             sem).wait()
        pltpu.sync_copy(rows_v, out_hbm.at[pl.ds(base, _CH_PER_W)])

    return gk(table, idx)


def kernel(z, embedding_weight):
    zt = jnp.transpose(z, (0, 2, 3, 1)).reshape(N_TOK, D)
    idx2, vq, commit, entl = _vq_main(zt, embedding_weight)
    idx = idx2.reshape(N_TOK)
    zq = _gather_rows(embedding_weight, idx.reshape(_N_CHUNKS, _CHUNK))
    zq = zq.reshape(8, 32, 32, D).transpose(0, 3, 1, 2)
    return zq, vq[0, 0], commit[0, 0], entl[0, 0], idx


# speedup vs baseline: 1.3338x; 1.3338x over previous
import functools

import jax
import jax.numpy as jnp
from jax import lax
from jax.experimental import pallas as pl
from jax.experimental.pallas import tpu as pltpu
from jax.experimental.pallas import tpu_sc as plsc

N_TOK = 8192
D = 256
K = 8192
R = 256
GRID = N_TOK // R
INV_TEMP = 100.0
LOG2E = 1.4426950408889634
LN2 = 0.6931471805599453

_NC = 2
_NS = 16
_NW = _NC * _NS
_CHUNK = 128
_N_CHUNKS = N_TOK // _CHUNK
_CH_PER_W = _N_CHUNKS // _NW


def _vq_body(z_ref, emb_ref, idx_ref, vq_ref, commit_ref, ent_ref,
             en_s, kio_s, colsum, vq_acc, ent_acc):
    i = pl.program_id(0)

    @pl.when(i == 0)
    def _init():
        colsum[...] = jnp.zeros_like(colsum)
        vq_acc[...] = jnp.zeros_like(vq_acc)
        ent_acc[...] = jnp.zeros_like(ent_acc)
        eb = emb_ref[...]
        enrm = jnp.sqrt(jnp.sum(eb * eb, axis=1, keepdims=True))
        en_s[...] = eb / (enrm + 1e-12)
        kio_s[...] = lax.broadcasted_iota(
            jnp.int32, (R, K), 1).astype(jnp.float32)

    zb = z_ref[...]
    nrm = jnp.sqrt(jnp.sum(zb * zb, axis=1, keepdims=True))
    zn = zb / (nrm + 1e-12)
    zsq = jnp.sum(zn * zn, axis=1, keepdims=True)

    dots = lax.dot_general(zn, en_s[...],
                           (((1,), (1,)), ((), ())),
                           preferred_element_type=jnp.float32)
    c = 2.0 - 2.0 * dots
    mc = jnp.min(c, axis=1, keepdims=True)
    idxf = jnp.min(jnp.where(c == mc, kio_s[...], float(K)), axis=1)
    idx_ref[...] = idxf.astype(jnp.int32).reshape(R, 1)

    t2 = c * (-INV_TEMP * LOG2E) + mc * (INV_TEMP * LOG2E)
    p = jnp.exp2(t2)
    pb = p.astype(jnp.bfloat16)
    ptb = (p * t2).astype(jnp.bfloat16)
    ones_k = jnp.ones((K, 1), jnp.bfloat16)
    zsum = lax.dot_general(pb, ones_k, (((1,), (0,)), ((), ())),
                           preferred_element_type=jnp.float32)
    invz_t = jnp.transpose(1.0 / zsum).astype(jnp.bfloat16)
    colsum[...] += lax.dot_general(invz_t, pb, (((1,), (0,)), ((), ())),
                                   preferred_element_type=jnp.float32)
    w2 = lax.dot_general(ptb, ones_k, (((1,), (0,)), ((), ())),
                         preferred_element_type=jnp.float32)
    ent_acc[...] += jnp.sum((LN2 * w2) / zsum - jnp.log(zsum),
                            axis=(0, 1), keepdims=True)
    vq_acc[...] += jnp.sum(zsq - 1.0 + mc, axis=(0, 1), keepdims=True)

    @pl.when(i == GRID - 1)
    def _fin():
        vq = vq_acc[...] * (1.0 / (N_TOK * D))
        vq_ref[...] = vq
        commit_ref[...] = 0.25 * vq
        ap = colsum[...] * (1.0 / N_TOK)
        avg_ent = -jnp.sum(ap * jnp.log(ap + 1e-5),
                           axis=(0, 1), keepdims=True)
        samp_ent = -ent_acc[...] * (1.0 / N_TOK)
        ent_ref[...] = 0.1 * (samp_ent - avg_ent)


def _vq_main(z_flat, emb):
    return pl.pallas_call(
        _vq_body,
        grid=(GRID,),
        in_specs=[
            pl.BlockSpec((R, D), lambda i: (i, 0)),
            pl.BlockSpec((K, D), lambda i: (0, 0)),
        ],
        out_specs=[
            pl.BlockSpec((R, 1), lambda i: (i, 0)),
            pl.BlockSpec((1, 1), lambda i: (0, 0)),
            pl.BlockSpec((1, 1), lambda i: (0, 0)),
            pl.BlockSpec((1, 1), lambda i: (0, 0)),
        ],
        out_shape=[
            jax.ShapeDtypeStruct((N_TOK, 1), jnp.int32),
            jax.ShapeDtypeStruct((1, 1), jnp.float32),
            jax.ShapeDtypeStruct((1, 1), jnp.float32),
            jax.ShapeDtypeStruct((1, 1), jnp.float32),
        ],
        scratch_shapes=[
            pltpu.VMEM((K, D), jnp.float32),
            pltpu.VMEM((R, K), jnp.float32),
            pltpu.VMEM((1, K), jnp.float32),
            pltpu.VMEM((1, 1), jnp.float32),
            pltpu.VMEM((1, 1), jnp.float32),
        ],
        compiler_params=pltpu.CompilerParams(
            dimension_semantics=("arbitrary",)),
    )(z_flat, emb)


def _gather_rows(table, idx):
    mesh = plsc.VectorSubcoreMesh(core_axis_name="c", subcore_axis_name="s")

    @functools.partial(
        pl.kernel,
        out_type=jax.ShapeDtypeStruct((_N_CHUNKS, _CHUNK, D), jnp.float32),
        mesh=mesh,
        scratch_types=[
            pltpu.VMEM((_CH_PER_W, _CHUNK), jnp.int32),
            pltpu.VMEM((_CH_PER_W, _CHUNK, D), jnp.float32),
            pltpu.SemaphoreType.DMA,
        ],
    )
    def gk(table_hbm, idx_hbm, out_hbm, idx_v, rows_v, sem):
        wid = lax.axis_index("s") * _NC + lax.axis_index("c")
        base = wid * _CH_PER_W
        pltpu.sync_copy(idx_hbm.at[pl.ds(base, _CH_PER_W)], idx_v)
        for j in range(_CH_PER_W):
            pltpu.async_copy(table_hbm.at[idx_v.at[j]], rows_v.at[j],
                             sem).wait()
        pltpu.sync_copy(rows_v, out_hbm.at[pl.ds(base, _CH_PER_W)])

    return gk(table, idx)


def kernel(z, embedding_weight):
    zt = jnp.transpose(z, (0, 2, 3, 1)).reshape(N_TOK, D)
    idx2, vq, commit, entl = _vq_main(zt, embedding_weight)
    idx = idx2.reshape(N_TOK)
    zq = _gather_rows(embedding_weight, idx.reshape(_N_CHUNKS, _CHUNK))
    zq = zq.reshape(8, 32, 32, D).transpose(0, 3, 1, 2)
    return zq, vq[0, 0], commit[0, 0], entl[0, 0], idx


# speedup vs baseline: 1.3360x; 1.0016x over previous
import functools

import jax
import jax.numpy as jnp
from jax import lax
from jax.experimental import pallas as pl
from jax.experimental.pallas import tpu as pltpu
from jax.experimental.pallas import tpu_sc as plsc

N_TOK = 8192
D = 256
K = 8192
R = 256
GRID = N_TOK // R
INV_TEMP = 100.0
LOG2E = 1.4426950408889634
LN2 = 0.6931471805599453

_NC = 2
_NS = 16
_NW = _NC * _NS
_CHUNK = 128
_N_CHUNKS = N_TOK // _CHUNK
_CH_PER_W = _N_CHUNKS // _NW


def _vq_body(z_ref, emb_ref, idx_ref, vq_ref, commit_ref, ent_ref,
             en_s, kio_s, colsum, vq_acc, ent_acc):
    i = pl.program_id(0)

    @pl.when(i == 0)
    def _init():
        colsum[...] = jnp.zeros_like(colsum)
        vq_acc[...] = jnp.zeros_like(vq_acc)
        ent_acc[...] = jnp.zeros_like(ent_acc)
        eb = emb_ref[...]
        enrm = jnp.sqrt(jnp.sum(eb * eb, axis=1, keepdims=True))
        en_s[...] = (eb / (enrm + 1e-12)).astype(jnp.bfloat16)
        kio_s[...] = lax.broadcasted_iota(
            jnp.int32, (1, K), 1).astype(jnp.float32)

    zb = z_ref[...]
    nrm = jnp.sqrt(jnp.sum(zb * zb, axis=1, keepdims=True))
    zn = zb / (nrm + 1e-12)
    zsq = jnp.sum(zn * zn, axis=1, keepdims=True)

    dots = lax.dot_general(zn.astype(jnp.bfloat16), en_s[...],
                           (((1,), (1,)), ((), ())),
                           preferred_element_type=jnp.float32)
    c = 2.0 - 2.0 * dots
    mc = jnp.min(c, axis=1, keepdims=True)
    idxf = jnp.min(jnp.where(c == mc, kio_s[...], float(K)), axis=1)
    idx_ref[...] = idxf.astype(jnp.int32).reshape(R, 1)

    t2 = c * (-INV_TEMP * LOG2E) + mc * (INV_TEMP * LOG2E)
    p = jnp.exp2(t2)
    pb = p.astype(jnp.bfloat16)
    ptb = (p * t2).astype(jnp.bfloat16)
    ones_k = jnp.ones((K, 1), jnp.bfloat16)
    zsum = lax.dot_general(pb, ones_k, (((1,), (0,)), ((), ())),
                           preferred_element_type=jnp.float32)
    invz_t = jnp.transpose(1.0 / zsum).astype(jnp.bfloat16)
    colsum[...] += lax.dot_general(invz_t, pb, (((1,), (0,)), ((), ())),
                                   preferred_element_type=jnp.float32)
    w2 = lax.dot_general(ptb, ones_k, (((1,), (0,)), ((), ())),
                         preferred_element_type=jnp.float32)
    ent_acc[...] += jnp.sum((LN2 * w2) / zsum - jnp.log(zsum),
                            axis=(0, 1), keepdims=True)
    vq_acc[...] += jnp.sum(zsq - 1.0 + mc, axis=(0, 1), keepdims=True)

    @pl.when(i == GRID - 1)
    def _fin():
        vq = vq_acc[...] * (1.0 / (N_TOK * D))
        vq_ref[...] = vq
        commit_ref[...] = 0.25 * vq
        ap = colsum[...] * (1.0 / N_TOK)
        avg_ent = -jnp.sum(ap * jnp.log(ap + 1e-5),
                           axis=(0, 1), keepdims=True)
        samp_ent = -ent_acc[...] * (1.0 / N_TOK)
        ent_ref[...] = 0.1 * (samp_ent - avg_ent)


def _vq_main(z_flat, emb):
    return pl.pallas_call(
        _vq_body,
        grid=(GRID,),
        in_specs=[
            pl.BlockSpec((R, D), lambda i: (i, 0)),
            pl.BlockSpec((K, D), lambda i: (0, 0)),
        ],
        out_specs=[
            pl.BlockSpec((R, 1), lambda i: (i, 0)),
            pl.BlockSpec((1, 1), lambda i: (0, 0)),
            pl.BlockSpec((1, 1), lambda i: (0, 0)),
            pl.BlockSpec((1, 1), lambda i: (0, 0)),
        ],
        out_shape=[
            jax.ShapeDtypeStruct((N_TOK, 1), jnp.int32),
            jax.ShapeDtypeStruct((1, 1), jnp.float32),
            jax.ShapeDtypeStruct((1, 1), jnp.float32),
            jax.ShapeDtypeStruct((1, 1), jnp.float32),
        ],
        scratch_shapes=[
            pltpu.VMEM((K, D), jnp.bfloat16),
            pltpu.VMEM((1, K), jnp.float32),
            pltpu.VMEM((1, K), jnp.float32),
            pltpu.VMEM((1, 1), jnp.float32),
            pltpu.VMEM((1, 1), jnp.float32),
        ],
        compiler_params=pltpu.CompilerParams(
            dimension_semantics=("arbitrary",)),
    )(z_flat, emb)


def _gather_rows(table, idx):
    mesh = plsc.VectorSubcoreMesh(core_axis_name="c", subcore_axis_name="s")

    @functools.partial(
        pl.kernel,
        out_type=jax.ShapeDtypeStruct((_N_CHUNKS, _CHUNK, D), jnp.float32),
        mesh=mesh,
        scratch_types=[
            pltpu.VMEM((_CH_PER_W, _CHUNK), jnp.int32),
            pltpu.VMEM((_CH_PER_W, _CHUNK, D), jnp.float32),
            pltpu.SemaphoreType.DMA,
        ],
    )
    def gk(table_hbm, idx_hbm, out_hbm, idx_v, rows_v, sem):
        wid = lax.axis_index("s") * _NC + lax.axis_index("c")
        base = wid * _CH_PER_W
        pltpu.sync_copy(idx_hbm.at[pl.ds(base, _CH_PER_W)], idx_v)
        for j in range(_CH_PER_W):
            pltpu.async_copy(table_hbm.at[idx_v.at[j]], rows_v.at[j],
                             sem).wait()
        pltpu.sync_copy(rows_v, out_hbm.at[pl.ds(base, _CH_PER_W)])

    return gk(table, idx)


def kernel(z, embedding_weight):
    zt = jnp.transpose(z, (0, 2, 3, 1)).reshape(N_TOK, D)
    idx2, vq, commit, entl = _vq_main(zt, embedding_weight)
    idx = idx2.reshape(N_TOK)
    zq = _gather_rows(embedding_weight, idx.reshape(_N_CHUNKS, _CHUNK))
    zq = zq.reshape(8, 32, 32, D).transpose(0, 3, 1, 2)
    return zq, vq[0, 0], commit[0, 0], entl[0, 0], idx
